# Initial kernel scaffold; baseline (speedup 1.0000x reference)
import functools

import jax
import jax.numpy as jnp
from jax import lax
from jax.experimental import pallas as pl
from jax.experimental.pallas import tpu as pltpu
from jax.experimental.pallas import tpu_sc as plsc

D = 16
CHUNK = 128
NC = 2
NS = 16
NW = NC * NS


def _proj_kernel(x_ref, w_ref, y_ref, r_ref):
    h = jnp.dot(x_ref[...], w_ref[...], preferred_element_type=jnp.float32)
    y_ref[...] = h[:, :D]
    r_ref[...] = h[:, D:]


def _mlp_kernel(part_ref, r_ref, bl_ref, w1_ref, b1_ref, w2_ref, b2_ref, o_ref):
    h = part_ref[0] + part_ref[1] + bl_ref[...] + r_ref[...]
    h = jnp.where(h >= 0, h, 0.01 * h)
    h = jnp.dot(h, w1_ref[...], preferred_element_type=jnp.float32) + b1_ref[...]
    h = jnp.where(h >= 0, h, 0.01 * h)
    o_ref[...] = jnp.dot(h, w2_ref[...], preferred_element_type=jnp.float32) + b2_ref[...]


def _make_agg(n_nodes, cpt, interpret=False):
    acc_rows = ((n_nodes + 1 + NS - 1) // NS) * NS
    zrows = acc_rows // NS
    drain = n_nodes // NS
    mesh = plsc.VectorSubcoreMesh(core_axis_name="c", subcore_axis_name="s")

    @functools.partial(
        pl.kernel,
        out_type=jax.ShapeDtypeStruct((NC, n_nodes, D), jnp.float32),
        mesh=mesh,
        scratch_types=[
            pltpu.VMEM((cpt, CHUNK), jnp.int32),
            pltpu.VMEM((cpt, CHUNK), jnp.int32),
            pltpu.VMEM((CHUNK, D), jnp.float32),
            pltpu.VMEM((zrows, D), jnp.float32),
            pltpu.VMEM_SHARED((acc_rows, D), jnp.float32),
            pltpu.SemaphoreType.DMA,
        ],
        interpret=interpret,
    )
    def agg(y_hbm, src_hbm, dst_hbm, out_hbm, src_v, dst_v, rows_v, zero_v,
            acc_sh, sem):
        c = lax.axis_index("c")
        s = lax.axis_index("s")
        wid = s * NC + c

        def zbody(i, carry):
            zero_v[i, :] = jnp.zeros((D,), jnp.float32)
            return carry

        lax.fori_loop(0, zrows, zbody, 0)
        pltpu.sync_copy(zero_v, acc_sh.at[pl.ds(s * zrows, zrows)])
        pltpu.sync_copy(src_hbm.at[pl.ds(wid * cpt, cpt)], src_v)
        pltpu.sync_copy(dst_hbm.at[pl.ds(wid * cpt, cpt)], dst_v)
        plsc.subcore_barrier()

        def body(j, carry):
            pltpu.async_copy(y_hbm.at[src_v.at[j]], rows_v, sem).wait()
            pltpu.sync_copy(rows_v, acc_sh.at[dst_v.at[j]], add=True)
            return carry

        lax.fori_loop(0, cpt, body, 0)
        plsc.subcore_barrier()
        pltpu.sync_copy(acc_sh.at[pl.ds(s * drain, drain)],
                        out_hbm.at[c, pl.ds(s * drain, drain)])

    return agg


def _run(x, edge_index, W_l, b_l, W_r, W1, b1, W2, b2, interpret=False):
    n_nodes, d_in = x.shape
    n_edges = edge_index.shape[1]
    cpt = -(-n_edges // (NW * CHUNK))
    e_pad = NW * cpt * CHUNK
    br = 1000 if n_nodes % 1000 == 0 else n_nodes

    src = edge_index[0].astype(jnp.int32)
    dst = edge_index[1].astype(jnp.int32)
    src = jnp.pad(src, (0, e_pad - n_edges)).reshape(NW * cpt, CHUNK)
    dst = jnp.pad(dst, (0, e_pad - n_edges),
                  constant_values=n_nodes).reshape(NW * cpt, CHUNK)

    w_cat = jnp.concatenate([W_l, W_r], axis=0).T
    grid = n_nodes // br
    y, r = pl.pallas_call(
        _proj_kernel,
        grid=(grid,),
        in_specs=[
            pl.BlockSpec((br, d_in), lambda i: (i, 0)),
            pl.BlockSpec((d_in, 2 * D), lambda i: (0, 0)),
        ],
        out_specs=[
            pl.BlockSpec((br, D), lambda i: (i, 0)),
            pl.BlockSpec((br, D), lambda i: (i, 0)),
        ],
        out_shape=[jax.ShapeDtypeStruct((n_nodes, D), jnp.float32)] * 2,
        interpret=interpret,
    )(x, w_cat)

    part = _make_agg(n_nodes, cpt, interpret=interpret)(y, src, dst)

    out = pl.pallas_call(
        _mlp_kernel,
        grid=(grid,),
        in_specs=[
            pl.BlockSpec((NC, br, D), lambda i: (0, i, 0)),
            pl.BlockSpec((br, D), lambda i: (i, 0)),
            pl.BlockSpec((1, D), lambda i: (0, 0)),
            pl.BlockSpec((D, D), lambda i: (0, 0)),
            pl.BlockSpec((1, D), lambda i: (0, 0)),
            pl.BlockSpec((D, D), lambda i: (0, 0)),
            pl.BlockSpec((1, D), lambda i: (0, 0)),
        ],
        out_specs=pl.BlockSpec((br, D), lambda i: (i, 0)),
        out_shape=jax.ShapeDtypeStruct((n_nodes, D), jnp.float32),
        interpret=interpret,
    )(part, r, b_l.reshape(1, D), W1.T, b1.reshape(1, D), W2.T,
      b2.reshape(1, D))
    return out


def kernel(x, edge_index, W_l, b_l, W_r, W1, b1, W2, b2):
    return _run(x, edge_index, W_l, b_l, W_r, W1, b1, W2, b2)


# speedup vs baseline: 10.6287x; 10.6287x over previous
import functools

import jax
import jax.numpy as jnp
from jax import lax
from jax.experimental import pallas as pl
from jax.experimental.pallas import tpu as pltpu
from jax.experimental.pallas import tpu_sc as plsc

D = 16
CHUNK = 128
NC = 2
NS = 16
NW = NC * NS


def _proj_kernel(x_ref, w_ref, y_ref, r_ref):
    h = jnp.dot(x_ref[...], w_ref[...], preferred_element_type=jnp.float32)
    y_ref[...] = h[:, :D]
    r_ref[...] = h[:, D:]


def _mlp_kernel(part_ref, r_ref, bl_ref, w1_ref, b1_ref, w2_ref, b2_ref, o_ref):
    h = part_ref[0] + part_ref[1] + bl_ref[...] + r_ref[...]
    h = jnp.where(h >= 0, h, 0.01 * h)
    h = jnp.dot(h, w1_ref[...], preferred_element_type=jnp.float32) + b1_ref[...]
    h = jnp.where(h >= 0, h, 0.01 * h)
    o_ref[...] = jnp.dot(h, w2_ref[...], preferred_element_type=jnp.float32) + b2_ref[...]


def _make_agg(n_nodes, cpt, interpret=False):
    acc_rows = ((n_nodes + 1 + 8 * NS - 1) // (8 * NS)) * (8 * NS)
    zrows = acc_rows // NS
    mesh = plsc.VectorSubcoreMesh(core_axis_name="c", subcore_axis_name="s",
                                  num_cores=NC, num_subcores=NS)

    @functools.partial(
        pl.kernel,
        out_type=jax.ShapeDtypeStruct((NC, acc_rows, D), jnp.float32),
        mesh=mesh,
        scratch_types=[
            pltpu.VMEM((cpt, CHUNK), jnp.int32),
            pltpu.VMEM((cpt, CHUNK), jnp.int32),
            pltpu.VMEM((CHUNK, D), jnp.float32),
            pltpu.VMEM((zrows, D), jnp.float32),
            pltpu.VMEM_SHARED((acc_rows, D), jnp.float32),
            pltpu.SemaphoreType.DMA,
        ],
        compiler_params=pltpu.CompilerParams(use_tc_tiling_on_sc=False),
        interpret=interpret,
    )
    def agg(y_hbm, src_hbm, dst_hbm, out_hbm, src_v, dst_v, rows_v, zero_v,
            acc_sh, sem):
        c = lax.axis_index("c")
        s = lax.axis_index("s")
        wid = s * NC + c

        def zbody(i, carry):
            zero_v[i, :] = jnp.zeros((D,), jnp.float32)
            return carry

        lax.fori_loop(0, zrows, zbody, 0)
        pltpu.sync_copy(zero_v, acc_sh.at[pl.ds(s * zrows, zrows)])
        pltpu.sync_copy(src_hbm.at[pl.ds(wid * cpt, cpt)], src_v)
        pltpu.sync_copy(dst_hbm.at[pl.ds(wid * cpt, cpt)], dst_v)
        plsc.subcore_barrier()

        def body(j, carry):
            pltpu.async_copy(y_hbm.at[src_v.at[j]], rows_v, sem).wait()
            pltpu.sync_copy(rows_v, acc_sh.at[dst_v.at[j]], add=True)
            return carry

        lax.fori_loop(0, cpt, body, 0)
        plsc.subcore_barrier()
        pltpu.sync_copy(acc_sh.at[pl.ds(s * zrows, zrows)],
                        out_hbm.at[c, pl.ds(s * zrows, zrows)])

    return agg


def _run(x, edge_index, W_l, b_l, W_r, W1, b1, W2, b2, interpret=False):
    n_nodes, d_in = x.shape
    n_edges = edge_index.shape[1]
    cpt = -(-n_edges // (NW * CHUNK))
    cpt = ((cpt + 7) // 8) * 8
    e_pad = NW * cpt * CHUNK
    br = 1000 if n_nodes % 1000 == 0 else n_nodes

    src = edge_index[0].astype(jnp.int32)
    dst = edge_index[1].astype(jnp.int32)
    src = jnp.pad(src, (0, e_pad - n_edges)).reshape(NW * cpt, CHUNK)
    dst = jnp.pad(dst, (0, e_pad - n_edges),
                  constant_values=n_nodes).reshape(NW * cpt, CHUNK)

    w_cat = jnp.concatenate([W_l, W_r], axis=0).T
    grid = n_nodes // br
    y, r = pl.pallas_call(
        _proj_kernel,
        grid=(grid,),
        in_specs=[
            pl.BlockSpec((br, d_in), lambda i: (i, 0)),
            pl.BlockSpec((d_in, 2 * D), lambda i: (0, 0)),
        ],
        out_specs=[
            pl.BlockSpec((br, D), lambda i: (i, 0)),
            pl.BlockSpec((br, D), lambda i: (i, 0)),
        ],
        out_shape=[jax.ShapeDtypeStruct((n_nodes, D), jnp.float32)] * 2,
        interpret=interpret,
    )(x, w_cat)

    part = _make_agg(n_nodes, cpt, interpret=interpret)(y, src, dst)

    out = pl.pallas_call(
        _mlp_kernel,
        grid=(grid,),
        in_specs=[
            pl.BlockSpec((NC, br, D), lambda i: (0, i, 0)),
            pl.BlockSpec((br, D), lambda i: (i, 0)),
            pl.BlockSpec((1, D), lambda i: (0, 0)),
            pl.BlockSpec((D, D), lambda i: (0, 0)),
            pl.BlockSpec((1, D), lambda i: (0, 0)),
            pl.BlockSpec((D, D), lambda i: (0, 0)),
            pl.BlockSpec((1, D), lambda i: (0, 0)),
        ],
        out_specs=pl.BlockSpec((br, D), lambda i: (i, 0)),
        out_shape=jax.ShapeDtypeStruct((n_nodes, D), jnp.float32),
        interpret=interpret,
    )(part, r, b_l.reshape(1, D), W1.T, b1.reshape(1, D), W2.T,
      b2.reshape(1, D))
    return out


def kernel(x, edge_index, W_l, b_l, W_r, W1, b1, W2, b2):
    return _run(x, edge_index, W_l, b_l, W_r, W1, b1, W2, b2)


# speedup vs baseline: 13.7255x; 1.2914x over previous
import functools

import jax
import jax.numpy as jnp
from jax import lax
from jax.experimental import pallas as pl
from jax.experimental.pallas import tpu as pltpu
from jax.experimental.pallas import tpu_sc as plsc

D = 16
CHUNK = 128
NC = 2
NS = 16
NW = NC * NS
NBUF = 4


def _proj_kernel(x_ref, w_ref, y_ref, r_ref):
    h = jnp.dot(x_ref[...], w_ref[...], preferred_element_type=jnp.float32)
    y_ref[...] = h[:, :D]
    r_ref[...] = h[:, D:]


def _mlp_kernel(part_ref, r_ref, bl_ref, w1_ref, b1_ref, w2_ref, b2_ref, o_ref):
    h = part_ref[0] + part_ref[1] + bl_ref[...] + r_ref[...]
    h = jnp.where(h >= 0, h, 0.01 * h)
    h = jnp.dot(h, w1_ref[...], preferred_element_type=jnp.float32) + b1_ref[...]
    h = jnp.where(h >= 0, h, 0.01 * h)
    o_ref[...] = jnp.dot(h, w2_ref[...], preferred_element_type=jnp.float32) + b2_ref[...]


def _make_agg(n_nodes, cpt, interpret=False):
    acc_rows = ((n_nodes + 1 + 8 * NS - 1) // (8 * NS)) * (8 * NS)
    zrows = acc_rows // NS
    mesh = plsc.VectorSubcoreMesh(core_axis_name="c", subcore_axis_name="s",
                                  num_cores=NC, num_subcores=NS)

    @functools.partial(
        pl.kernel,
        out_type=jax.ShapeDtypeStruct((NC, acc_rows, D), jnp.float32),
        mesh=mesh,
        scratch_types=[
            pltpu.VMEM((cpt, CHUNK), jnp.int32),
            pltpu.VMEM((cpt, CHUNK), jnp.int32),
            pltpu.VMEM((NBUF, CHUNK, D), jnp.float32),
            pltpu.VMEM((zrows, D), jnp.float32),
            pltpu.VMEM_SHARED((acc_rows, D), jnp.float32),
            [pltpu.SemaphoreType.DMA] * NBUF,
        ],
        compiler_params=pltpu.CompilerParams(use_tc_tiling_on_sc=False),
        interpret=interpret,
    )
    def agg(y_hbm, src_hbm, dst_hbm, out_hbm, src_v, dst_v, rows_v, zero_v,
            acc_sh, sems):
        c = lax.axis_index("c")
        s = lax.axis_index("s")
        wid = s * NC + c

        def zbody(i, carry):
            zero_v[i, :] = jnp.zeros((D,), jnp.float32)
            return carry

        lax.fori_loop(0, zrows, zbody, 0)
        pltpu.sync_copy(zero_v, acc_sh.at[pl.ds(s * zrows, zrows)])
        pltpu.sync_copy(src_hbm.at[pl.ds(wid * cpt, cpt)], src_v)
        pltpu.sync_copy(dst_hbm.at[pl.ds(wid * cpt, cpt)], dst_v)
        plsc.subcore_barrier()

        for b in range(NBUF):
            pltpu.async_copy(y_hbm.at[src_v.at[b]], rows_v.at[b], sems[b])

        def body(g, carry):
            base = g * NBUF
            for b in range(NBUF):
                j = base + b
                pltpu.make_async_copy(y_hbm.at[src_v.at[j]], rows_v.at[b],
                                      sems[b]).wait()
                pltpu.sync_copy(rows_v.at[b], acc_sh.at[dst_v.at[j]], add=True)

                @pl.when(j + NBUF < cpt)
                def _():
                    pltpu.async_copy(y_hbm.at[src_v.at[j + NBUF]],
                                     rows_v.at[b], sems[b])
            return carry

        lax.fori_loop(0, cpt // NBUF, body, 0)
        plsc.subcore_barrier()
        pltpu.sync_copy(acc_sh.at[pl.ds(s * zrows, zrows)],
                        out_hbm.at[c, pl.ds(s * zrows, zrows)])

    return agg


def _run(x, edge_index, W_l, b_l, W_r, W1, b1, W2, b2, interpret=False):
    n_nodes, d_in = x.shape
    n_edges = edge_index.shape[1]
    cpt = -(-n_edges // (NW * CHUNK))
    cpt = ((cpt + 7) // 8) * 8
    e_pad = NW * cpt * CHUNK
    br = 1000 if n_nodes % 1000 == 0 else n_nodes

    src = edge_index[0].astype(jnp.int32)
    dst = edge_index[1].astype(jnp.int32)
    src = jnp.pad(src, (0, e_pad - n_edges)).reshape(NW * cpt, CHUNK)
    dst = jnp.pad(dst, (0, e_pad - n_edges),
                  constant_values=n_nodes).reshape(NW * cpt, CHUNK)

    w_cat = jnp.concatenate([W_l, W_r], axis=0).T
    grid = n_nodes // br
    y, r = pl.pallas_call(
        _proj_kernel,
        grid=(grid,),
        in_specs=[
            pl.BlockSpec((br, d_in), lambda i: (i, 0)),
            pl.BlockSpec((d_in, 2 * D), lambda i: (0, 0)),
        ],
        out_specs=[
            pl.BlockSpec((br, D), lambda i: (i, 0)),
            pl.BlockSpec((br, D), lambda i: (i, 0)),
        ],
        out_shape=[jax.ShapeDtypeStruct((n_nodes, D), jnp.float32)] * 2,
        interpret=interpret,
    )(x, w_cat)

    part = _make_agg(n_nodes, cpt, interpret=interpret)(y, src, dst)

    out = pl.pallas_call(
        _mlp_kernel,
        grid=(grid,),
        in_specs=[
            pl.BlockSpec((NC, br, D), lambda i: (0, i, 0)),
            pl.BlockSpec((br, D), lambda i: (i, 0)),
            pl.BlockSpec((1, D), lambda i: (0, 0)),
            pl.BlockSpec((D, D), lambda i: (0, 0)),
            pl.BlockSpec((1, D), lambda i: (0, 0)),
            pl.BlockSpec((D, D), lambda i: (0, 0)),
            pl.BlockSpec((1, D), lambda i: (0, 0)),
        ],
        out_specs=pl.BlockSpec((br, D), lambda i: (i, 0)),
        out_shape=jax.ShapeDtypeStruct((n_nodes, D), jnp.float32),
        interpret=interpret,
    )(part, r, b_l.reshape(1, D), W1.T, b1.reshape(1, D), W2.T,
      b2.reshape(1, D))
    return out


def kernel(x, edge_index, W_l, b_l, W_r, W1, b1, W2, b2):
    return _run(x, edge_index, W_l, b_l, W_r, W1, b1, W2, b2)


# speedup vs baseline: 13.8895x; 1.0119x over previous
import functools

import jax
import jax.numpy as jnp
from jax import lax
from jax.experimental import pallas as pl
from jax.experimental.pallas import tpu as pltpu
from jax.experimental.pallas import tpu_sc as plsc

D = 16
CHUNK = 128
NC = 2
NS = 16
NW = NC * NS
NBUF = 8
LA = 6


def _proj_kernel(x_ref, w_ref, y_ref, r_ref):
    h = jnp.dot(x_ref[...], w_ref[...], preferred_element_type=jnp.float32)
    y_ref[...] = h[:, :D]
    r_ref[...] = h[:, D:]


def _mlp_kernel(part_ref, r_ref, bl_ref, w1_ref, b1_ref, w2_ref, b2_ref, o_ref):
    h = part_ref[0] + part_ref[1] + bl_ref[...] + r_ref[...]
    h = jnp.where(h >= 0, h, 0.01 * h)
    h = jnp.dot(h, w1_ref[...], preferred_element_type=jnp.float32) + b1_ref[...]
    h = jnp.where(h >= 0, h, 0.01 * h)
    o_ref[...] = jnp.dot(h, w2_ref[...], preferred_element_type=jnp.float32) + b2_ref[...]


def _make_agg(n_nodes, cpt, interpret=False):
    acc_rows = ((n_nodes + 1 + 8 * NS - 1) // (8 * NS)) * (8 * NS)
    zrows = acc_rows // NS
    mesh = plsc.VectorSubcoreMesh(core_axis_name="c", subcore_axis_name="s",
                                  num_cores=NC, num_subcores=NS)

    @functools.partial(
        pl.kernel,
        out_type=jax.ShapeDtypeStruct((NC, acc_rows, D), jnp.float32),
        mesh=mesh,
        scratch_types=[
            pltpu.VMEM((cpt, CHUNK), jnp.int32),
            pltpu.VMEM((cpt, CHUNK), jnp.int32),
            pltpu.VMEM((NBUF, CHUNK, D), jnp.float32),
            pltpu.VMEM((zrows, D), jnp.float32),
            pltpu.VMEM_SHARED((acc_rows, D), jnp.float32),
            [pltpu.SemaphoreType.DMA] * NBUF,
            [pltpu.SemaphoreType.DMA] * NBUF,
        ],
        compiler_params=pltpu.CompilerParams(use_tc_tiling_on_sc=False),
        interpret=interpret,
    )
    def agg(y_hbm, src_hbm, dst_hbm, out_hbm, src_v, dst_v, rows_v, zero_v,
            acc_sh, gsems, ssems):
        c = lax.axis_index("c")
        s = lax.axis_index("s")
        wid = s * NC + c

        def zbody(i, carry):
            zero_v[i, :] = jnp.zeros((D,), jnp.float32)
            return carry

        lax.fori_loop(0, zrows, zbody, 0)
        pltpu.sync_copy(zero_v, acc_sh.at[pl.ds(s * zrows, zrows)])
        pltpu.sync_copy(src_hbm.at[pl.ds(wid * cpt, cpt)], src_v)
        pltpu.sync_copy(dst_hbm.at[pl.ds(wid * cpt, cpt)], dst_v)
        plsc.subcore_barrier()

        for b in range(LA):
            pltpu.async_copy(y_hbm.at[src_v.at[b]], rows_v.at[b], gsems[b])

        def body(g, carry):
            base = g * NBUF
            for b in range(NBUF):
                j = base + b
                bf = (b + LA) % NBUF
                pltpu.make_async_copy(y_hbm.at[src_v.at[j]], rows_v.at[b],
                                      gsems[b]).wait()
                pltpu.async_copy(rows_v.at[b], acc_sh.at[dst_v.at[j]],
                                 ssems[b], add=True)
                f = j + LA

                @pl.when(f < cpt)
                def _():
                    @pl.when(f >= NBUF)
                    def _():
                        pltpu.make_async_copy(
                            rows_v.at[bf], acc_sh.at[dst_v.at[f - NBUF]],
                            ssems[bf]).wait()

                    pltpu.async_copy(y_hbm.at[src_v.at[f]], rows_v.at[bf],
                                     gsems[bf])
            return carry

        lax.fori_loop(0, cpt // NBUF, body, 0)
        for j in range(cpt - NBUF, cpt):
            b = j % NBUF
            pltpu.make_async_copy(rows_v.at[b], acc_sh.at[dst_v.at[j]],
                                  ssems[b]).wait()
        plsc.subcore_barrier()
        pltpu.sync_copy(acc_sh.at[pl.ds(s * zrows, zrows)],
                        out_hbm.at[c, pl.ds(s * zrows, zrows)])

    return agg


def _run(x, edge_index, W_l, b_l, W_r, W1, b1, W2, b2, interpret=False):
    n_nodes, d_in = x.shape
    n_edges = edge_index.shape[1]
    cpt = -(-n_edges // (NW * CHUNK))
    cpt = ((cpt + 7) // 8) * 8
    e_pad = NW * cpt * CHUNK
    br = 1000 if n_nodes % 1000 == 0 else n_nodes

    src = edge_index[0].astype(jnp.int32)
    dst = edge_index[1].astype(jnp.int32)
    src = jnp.pad(src, (0, e_pad - n_edges)).reshape(NW * cpt, CHUNK)
    dst = jnp.pad(dst, (0, e_pad - n_edges),
                  constant_values=n_nodes).reshape(NW * cpt, CHUNK)

    w_cat = jnp.concatenate([W_l, W_r], axis=0).T
    grid = n_nodes // br
    y, r = pl.pallas_call(
        _proj_kernel,
        grid=(grid,),
        in_specs=[
            pl.BlockSpec((br, d_in), lambda i: (i, 0)),
            pl.BlockSpec((d_in, 2 * D), lambda i: (0, 0)),
        ],
        out_specs=[
            pl.BlockSpec((br, D), lambda i: (i, 0)),
            pl.BlockSpec((br, D), lambda i: (i, 0)),
        ],
        out_shape=[jax.ShapeDtypeStruct((n_nodes, D), jnp.float32)] * 2,
        interpret=interpret,
    )(x, w_cat)

    part = _make_agg(n_nodes, cpt, interpret=interpret)(y, src, dst)

    out = pl.pallas_call(
        _mlp_kernel,
        grid=(grid,),
        in_specs=[
            pl.BlockSpec((NC, br, D), lambda i: (0, i, 0)),
            pl.BlockSpec((br, D), lambda i: (i, 0)),
            pl.BlockSpec((1, D), lambda i: (0, 0)),
            pl.BlockSpec((D, D), lambda i: (0, 0)),
            pl.BlockSpec((1, D), lambda i: (0, 0)),
            pl.BlockSpec((D, D), lambda i: (0, 0)),
            pl.BlockSpec((1, D), lambda i: (0, 0)),
        ],
        out_specs=pl.BlockSpec((br, D), lambda i: (i, 0)),
        out_shape=jax.ShapeDtypeStruct((n_nodes, D), jnp.float32),
        interpret=interpret,
    )(part, r, b_l.reshape(1, D), W1.T, b1.reshape(1, D), W2.T,
      b2.reshape(1, D))
    return out


def kernel(x, edge_index, W_l, b_l, W_r, W1, b1, W2, b2):
    return _run(x, edge_index, W_l, b_l, W_r, W1, b1, W2, b2)


# speedup vs baseline: 21.9239x; 1.5784x over previous
import functools

import jax
import jax.numpy as jnp
from jax import lax
from jax.experimental import pallas as pl
from jax.experimental.pallas import tpu as pltpu
from jax.experimental.pallas import tpu_sc as plsc

D = 16
CHUNK = 128
NC = 2
NS = 16
NW = NC * NS
NBUF = 8
LA = 6


def _proj_kernel(x_ref, w_ref, y_ref, r_ref):
    h = jnp.dot(x_ref[...], w_ref[...], preferred_element_type=jnp.float32)
    y_ref[...] = h[:, :D]
    r_ref[...] = h[:, D:]


def _mlp_kernel(part_ref, r_ref, bl_ref, w1_ref, b1_ref, w2_ref, b2_ref, o_ref):
    h = part_ref[0] + part_ref[1] + bl_ref[...] + r_ref[...]
    h = jnp.where(h >= 0, h, 0.01 * h)
    h = jnp.dot(h, w1_ref[...], preferred_element_type=jnp.float32) + b1_ref[...]
    h = jnp.where(h >= 0, h, 0.01 * h)
    o_ref[...] = jnp.dot(h, w2_ref[...], preferred_element_type=jnp.float32) + b2_ref[...]


def _acc_rows(n_nodes):
    return ((n_nodes + 1 + 8 * NS - 1) // (8 * NS)) * (8 * NS)


def _make_agg(n_nodes, cpt, interpret=False):
    acc_rows = _acc_rows(n_nodes)
    zrows = acc_rows // NS
    mesh = plsc.VectorSubcoreMesh(core_axis_name="c", subcore_axis_name="s",
                                  num_cores=NC, num_subcores=NS)

    @functools.partial(
        pl.kernel,
        out_type=jax.ShapeDtypeStruct((NC, acc_rows, D), jnp.float32),
        mesh=mesh,
        scratch_types=[
            pltpu.VMEM((cpt, CHUNK), jnp.int32),
            pltpu.VMEM((cpt, CHUNK), jnp.int32),
            pltpu.VMEM((NBUF, CHUNK, D), jnp.float32),
            pltpu.VMEM((zrows, D), jnp.float32),
            pltpu.VMEM_SHARED((acc_rows, D), jnp.float32),
            [pltpu.SemaphoreType.DMA] * NBUF,
            [pltpu.SemaphoreType.DMA] * NBUF,
        ],
        compiler_params=pltpu.CompilerParams(use_tc_tiling_on_sc=False),
        interpret=interpret,
    )
    def agg(y_hbm, src_hbm, dst_hbm, out_hbm, src_v, dst_v, rows_v, zero_v,
            acc_sh, gsems, ssems):
        c = lax.axis_index("c")
        s = lax.axis_index("s")
        wid = s * NC + c

        def zbody(i, carry):
            zero_v[i, :] = jnp.zeros((D,), jnp.float32)
            return carry

        lax.fori_loop(0, zrows, zbody, 0)
        pltpu.sync_copy(zero_v, acc_sh.at[pl.ds(s * zrows, zrows)])
        pltpu.sync_copy(src_hbm.at[pl.ds(wid * cpt, cpt)], src_v)
        pltpu.sync_copy(dst_hbm.at[pl.ds(wid * cpt, cpt)], dst_v)
        plsc.subcore_barrier()

        for b in range(LA):
            pltpu.async_copy(y_hbm.at[src_v.at[b]], rows_v.at[b], gsems[b])

        def body(g, carry):
            base = g * NBUF
            for b in range(NBUF):
                j = base + b
                bf = (b + LA) % NBUF
                pltpu.make_async_copy(y_hbm.at[src_v.at[j]], rows_v.at[b],
                                      gsems[b]).wait()
                pltpu.async_copy(rows_v.at[b], acc_sh.at[dst_v.at[j]],
                                 ssems[b], add=True)
                f = j + LA

                @pl.when(f < cpt)
                def _():
                    @pl.when(f >= NBUF)
                    def _():
                        pltpu.make_async_copy(
                            rows_v.at[bf], acc_sh.at[dst_v.at[f - NBUF]],
                            ssems[bf]).wait()

                    pltpu.async_copy(y_hbm.at[src_v.at[f]], rows_v.at[bf],
                                     gsems[bf])
            return carry

        lax.fori_loop(0, cpt // NBUF, body, 0)
        for j in range(cpt - NBUF, cpt):
            b = j % NBUF
            pltpu.make_async_copy(rows_v.at[b], acc_sh.at[dst_v.at[j]],
                                  ssems[b]).wait()
        plsc.subcore_barrier()
        pltpu.sync_copy(acc_sh.at[pl.ds(s * zrows, zrows)],
                        out_hbm.at[c, pl.ds(s * zrows, zrows)])

    return agg


def _run(x, edge_index, W_l, b_l, W_r, W1, b1, W2, b2, interpret=False):
    n_nodes, d_in = x.shape
    n_edges = edge_index.shape[1]
    cpt = -(-n_edges // (NW * CHUNK))
    cpt = ((cpt + 7) // 8) * 8
    e_pad = NW * cpt * CHUNK
    br = n_nodes

    npad = e_pad - n_edges
    junk = _acc_rows(n_nodes) - n_nodes
    pad_src = (jnp.arange(npad, dtype=jnp.int32) * 37) % n_nodes
    pad_dst = n_nodes + (jnp.arange(npad, dtype=jnp.int32) % junk)
    src = jnp.concatenate([edge_index[0].astype(jnp.int32), pad_src])
    dst = jnp.concatenate([edge_index[1].astype(jnp.int32), pad_dst])
    src = src.reshape(NW * cpt, CHUNK)
    dst = dst.reshape(NW * cpt, CHUNK)

    w_cat = jnp.concatenate([W_l, W_r], axis=0).T
    grid = n_nodes // br
    y, r = pl.pallas_call(
        _proj_kernel,
        grid=(grid,),
        in_specs=[
            pl.BlockSpec((br, d_in), lambda i: (i, 0)),
            pl.BlockSpec((d_in, 2 * D), lambda i: (0, 0)),
        ],
        out_specs=[
            pl.BlockSpec((br, D), lambda i: (i, 0)),
            pl.BlockSpec((br, D), lambda i: (i, 0)),
        ],
        out_shape=[jax.ShapeDtypeStruct((n_nodes, D), jnp.float32)] * 2,
        interpret=interpret,
    )(x, w_cat)

    part = _make_agg(n_nodes, cpt, interpret=interpret)(y, src, dst)

    out = pl.pallas_call(
        _mlp_kernel,
        grid=(grid,),
        in_specs=[
            pl.BlockSpec((NC, br, D), lambda i: (0, i, 0)),
            pl.BlockSpec((br, D), lambda i: (i, 0)),
            pl.BlockSpec((1, D), lambda i: (0, 0)),
            pl.BlockSpec((D, D), lambda i: (0, 0)),
            pl.BlockSpec((1, D), lambda i: (0, 0)),
            pl.BlockSpec((D, D), lambda i: (0, 0)),
            pl.BlockSpec((1, D), lambda i: (0, 0)),
        ],
        out_specs=pl.BlockSpec((br, D), lambda i: (i, 0)),
        out_shape=jax.ShapeDtypeStruct((n_nodes, D), jnp.float32),
        interpret=interpret,
    )(part, r, b_l.reshape(1, D), W1.T, b1.reshape(1, D), W2.T,
      b2.reshape(1, D))
    return out


def kernel(x, edge_index, W_l, b_l, W_r, W1, b1, W2, b2):
    return _run(x, edge_index, W_l, b_l, W_r, W1, b1, W2, b2)


# speedup vs baseline: 23.3081x; 1.0631x over previous
import functools

import jax
import jax.numpy as jnp
from jax import lax
from jax.experimental import pallas as pl
from jax.experimental.pallas import tpu as pltpu
from jax.experimental.pallas import tpu_sc as plsc

D = 16
CHUNK = 128
NC = 2
NS = 16
NW = NC * NS
NBUF = 6
LA = 4


def _proj_kernel(x_ref, w_ref, y_ref, r_ref):
    h = jnp.dot(x_ref[...], w_ref[...], preferred_element_type=jnp.float32)
    y_ref[...] = h[:, :D]
    r_ref[...] = h[:, D:]


def _mlp_kernel(part_ref, r_ref, bl_ref, w1_ref, b1_ref, w2_ref, b2_ref, o_ref):
    h = part_ref[0] + part_ref[1] + bl_ref[...] + r_ref[...]
    h = jnp.where(h >= 0, h, 0.01 * h)
    h = jnp.dot(h, w1_ref[...], preferred_element_type=jnp.float32) + b1_ref[...]
    h = jnp.where(h >= 0, h, 0.01 * h)
    o_ref[...] = jnp.dot(h, w2_ref[...], preferred_element_type=jnp.float32) + b2_ref[...]


def _make_agg(n_nodes, n_edges, interpret=False):
    acc_rows = ((n_nodes + 8 * NS - 1) // (8 * NS)) * (8 * NS)
    zrows = acc_rows // NS
    last = n_nodes - (NS - 1) * zrows
    assert 0 < last <= zrows
    assert n_edges % NW == 0
    ept = n_edges // NW
    cptf = ept // CHUNK
    tail = ept - cptf * CHUNK
    assert cptf % NBUF == 0 and cptf >= NBUF
    mesh = plsc.VectorSubcoreMesh(core_axis_name="c", subcore_axis_name="s",
                                  num_cores=NC, num_subcores=NS)

    @functools.partial(
        pl.kernel,
        out_type=jax.ShapeDtypeStruct((NC, n_nodes, D), jnp.float32),
        mesh=mesh,
        scratch_types=[
            pltpu.VMEM((ept,), jnp.int32),
            pltpu.VMEM((ept,), jnp.int32),
            pltpu.VMEM((NBUF, CHUNK, D), jnp.float32),
            pltpu.VMEM((zrows, D), jnp.float32),
            pltpu.VMEM_SHARED((acc_rows, D), jnp.float32),
            [pltpu.SemaphoreType.DMA] * NBUF,
            [pltpu.SemaphoreType.DMA] * NBUF,
        ],
        compiler_params=pltpu.CompilerParams(use_tc_tiling_on_sc=False),
        interpret=interpret,
    )
    def agg(y_hbm, ei_hbm, out_hbm, src_v, dst_v, rows_v, zero_v,
            acc_sh, gsems, ssems):
        c = lax.axis_index("c")
        s = lax.axis_index("s")
        wid = s * NC + c

        def zbody(i, carry):
            zero_v[i, :] = jnp.zeros((D,), jnp.float32)
            return carry

        lax.fori_loop(0, zrows, zbody, 0)
        pltpu.sync_copy(zero_v, acc_sh.at[pl.ds(s * zrows, zrows)])
        pltpu.sync_copy(ei_hbm.at[0, pl.ds(wid * ept, ept)], src_v)
        pltpu.sync_copy(ei_hbm.at[1, pl.ds(wid * ept, ept)], dst_v)
        plsc.subcore_barrier()

        def sidx(j):
            return src_v.at[pl.ds(j * CHUNK, CHUNK)]

        def didx(j):
            return dst_v.at[pl.ds(j * CHUNK, CHUNK)]

        for b in range(LA):
            pltpu.async_copy(y_hbm.at[sidx(b)], rows_v.at[b], gsems[b])

        def body(g, carry):
            base = g * NBUF
            for b in range(NBUF):
                j = base + b
                bf = (b + LA) % NBUF
                pltpu.make_async_copy(y_hbm.at[sidx(j)], rows_v.at[b],
                                      gsems[b]).wait()
                pltpu.async_copy(rows_v.at[b], acc_sh.at[didx(j)],
                                 ssems[b], add=True)
                f = j + LA

                @pl.when(f < cptf)
                def _():
                    @pl.when(f >= NBUF)
                    def _():
                        pltpu.make_async_copy(
                            rows_v.at[bf], acc_sh.at[didx(f - NBUF)],
                            ssems[bf]).wait()

                    pltpu.async_copy(y_hbm.at[sidx(f)], rows_v.at[bf],
                                     gsems[bf])
            return carry

        lax.fori_loop(0, cptf // NBUF, body, 0)
        for j in range(cptf - NBUF, cptf):
            b = j % NBUF
            pltpu.make_async_copy(rows_v.at[b], acc_sh.at[didx(j)],
                                  ssems[b]).wait()
        if tail:
            tbase = cptf * CHUNK
            pltpu.sync_copy(y_hbm.at[src_v.at[pl.ds(tbase, tail)]],
                            rows_v.at[0, pl.ds(0, tail)])
            pltpu.sync_copy(rows_v.at[0, pl.ds(0, tail)],
                            acc_sh.at[dst_v.at[pl.ds(tbase, tail)]], add=True)
        plsc.subcore_barrier()

        @pl.when(s < NS - 1)
        def _():
            pltpu.sync_copy(acc_sh.at[pl.ds(s * zrows, zrows)],
                            out_hbm.at[c, pl.ds(s * zrows, zrows)])

        @pl.when(s == NS - 1)
        def _():
            pltpu.sync_copy(acc_sh.at[pl.ds((NS - 1) * zrows, last)],
                            out_hbm.at[c, pl.ds((NS - 1) * zrows, last)])

    return agg


def _run(x, edge_index, W_l, b_l, W_r, W1, b1, W2, b2, interpret=False):
    n_nodes, d_in = x.shape
    n_edges = edge_index.shape[1]
    br = n_nodes

    w_cat = jnp.concatenate([W_l, W_r], axis=0).T
    grid = n_nodes // br
    y, r = pl.pallas_call(
        _proj_kernel,
        grid=(grid,),
        in_specs=[
            pl.BlockSpec((br, d_in), lambda i: (i, 0)),
            pl.BlockSpec((d_in, 2 * D), lambda i: (0, 0)),
        ],
        out_specs=[
            pl.BlockSpec((br, D), lambda i: (i, 0)),
            pl.BlockSpec((br, D), lambda i: (i, 0)),
        ],
        out_shape=[jax.ShapeDtypeStruct((n_nodes, D), jnp.float32)] * 2,
        interpret=interpret,
    )(x, w_cat)

    part = _make_agg(n_nodes, n_edges, interpret=interpret)(
        y, edge_index.astype(jnp.int32))

    out = pl.pallas_call(
        _mlp_kernel,
        grid=(grid,),
        in_specs=[
            pl.BlockSpec((NC, br, D), lambda i: (0, i, 0)),
            pl.BlockSpec((br, D), lambda i: (i, 0)),
            pl.BlockSpec((1, D), lambda i: (0, 0)),
            pl.BlockSpec((D, D), lambda i: (0, 0)),
            pl.BlockSpec((1, D), lambda i: (0, 0)),
            pl.BlockSpec((D, D), lambda i: (0, 0)),
            pl.BlockSpec((1, D), lambda i: (0, 0)),
        ],
        out_specs=pl.BlockSpec((br, D), lambda i: (i, 0)),
        out_shape=jax.ShapeDtypeStruct((n_nodes, D), jnp.float32),
        interpret=interpret,
    )(part, r, b_l.reshape(1, D), W1.T, b1.reshape(1, D), W2.T,
      b2.reshape(1, D))
    return out


def kernel(x, edge_index, W_l, b_l, W_r, W1, b1, W2, b2):
    return _run(x, edge_index, W_l, b_l, W_r, W1, b1, W2, b2)


# speedup vs baseline: 25.1157x; 1.0776x over previous
import functools

import jax
import jax.numpy as jnp
from jax import lax
from jax.experimental import pallas as pl
from jax.experimental.pallas import tpu as pltpu
from jax.experimental.pallas import tpu_sc as plsc

D = 16
CHUNK = 128
NC = 2
NS = 16
NW = NC * NS
NBUF = 6
LA = 4


def _proj_kernel(x_ref, w_ref, y_ref, r_ref):
    h = jnp.dot(x_ref[...], w_ref[...], preferred_element_type=jnp.float32)
    y_ref[...] = h[:, :D]
    r_ref[...] = h[:, D:]


def _mlp_kernel(part_ref, r_ref, bl_ref, w1_ref, b1_ref, w2_ref, b2_ref, o_ref):
    h = part_ref[0] + part_ref[1] + bl_ref[...] + r_ref[...]
    h = jnp.where(h >= 0, h, 0.01 * h)
    h = jnp.dot(h, w1_ref[...], preferred_element_type=jnp.float32) + b1_ref[...]
    h = jnp.where(h >= 0, h, 0.01 * h)
    o_ref[...] = jnp.dot(h, w2_ref[...], preferred_element_type=jnp.float32) + b2_ref[...]


def _make_agg(n_nodes, n_edges, interpret=False):
    acc_rows = ((n_nodes + 8 * NS - 1) // (8 * NS)) * (8 * NS)
    zrows = acc_rows // NS
    last = n_nodes - (NS - 1) * zrows
    assert 0 < last <= zrows
    assert n_edges % CHUNK == 0
    nrows = n_edges // CHUNK
    base_cpt = nrows // NW
    extra = nrows - base_cpt * NW
    ngrp = base_cpt // NBUF
    rest = base_cpt - ngrp * NBUF
    assert ngrp >= 1
    mesh = plsc.VectorSubcoreMesh(core_axis_name="c", subcore_axis_name="s",
                                  num_cores=NC, num_subcores=NS)

    @functools.partial(
        pl.kernel,
        out_type=jax.ShapeDtypeStruct((NC, n_nodes, D), jnp.float32),
        mesh=mesh,
        scratch_types=[
            pltpu.VMEM((base_cpt + 1, 2, CHUNK), jnp.int32),
            pltpu.VMEM((NBUF, CHUNK, D), jnp.float32),
            pltpu.VMEM((zrows, D), jnp.float32),
            pltpu.VMEM_SHARED((acc_rows, D), jnp.float32),
            [pltpu.SemaphoreType.DMA] * NBUF,
            [pltpu.SemaphoreType.DMA] * NBUF,
        ],
        compiler_params=pltpu.CompilerParams(use_tc_tiling_on_sc=False),
        interpret=interpret,
    )
    def agg(y_hbm, ei_hbm, out_hbm, ei_v, rows_v, zero_v, acc_sh,
            gsems, ssems):
        c = lax.axis_index("c")
        s = lax.axis_index("s")
        wid = s * NC + c
        start = wid * base_cpt + jnp.minimum(wid, extra)

        def zbody(i, carry):
            zero_v[i, :] = jnp.zeros((D,), jnp.float32)
            return carry

        lax.fori_loop(0, zrows, zbody, 0)
        pltpu.sync_copy(zero_v, acc_sh.at[pl.ds(s * zrows, zrows)])

        @pl.when(wid < extra)
        def _():
            pltpu.sync_copy(ei_hbm.at[pl.ds(start, base_cpt + 1)], ei_v)

        @pl.when(wid >= extra)
        def _():
            pltpu.sync_copy(ei_hbm.at[pl.ds(start, base_cpt)],
                            ei_v.at[pl.ds(0, base_cpt)])

        plsc.subcore_barrier()

        for b in range(LA):
            pltpu.async_copy(y_hbm.at[ei_v.at[b, 0]], rows_v.at[b], gsems[b])

        def step(j, b, bf):
            pltpu.make_async_copy(y_hbm.at[ei_v.at[j, 0]], rows_v.at[b],
                                  gsems[b]).wait()
            pltpu.async_copy(rows_v.at[b], acc_sh.at[ei_v.at[j, 1]],
                             ssems[b], add=True)
            f = j + LA

            @pl.when(f < base_cpt)
            def _():
                @pl.when(f >= NBUF)
                def _():
                    pltpu.make_async_copy(
                        rows_v.at[bf], acc_sh.at[ei_v.at[f - NBUF, 1]],
                        ssems[bf]).wait()

                pltpu.async_copy(y_hbm.at[ei_v.at[f, 0]], rows_v.at[bf],
                                 gsems[bf])

        def body(g, carry):
            base = g * NBUF
            for b in range(NBUF):
                step(base + b, b, (b + LA) % NBUF)
            return carry

        lax.fori_loop(0, ngrp, body, 0)
        for j in range(ngrp * NBUF, base_cpt):
            step(j, j % NBUF, (j + LA) % NBUF)
        for j in range(base_cpt - NBUF, base_cpt):
            b = j % NBUF
            pltpu.make_async_copy(rows_v.at[b], acc_sh.at[ei_v.at[j, 1]],
                                  ssems[b]).wait()

        @pl.when(wid < extra)
        def _():
            pltpu.sync_copy(y_hbm.at[ei_v.at[base_cpt, 0]], rows_v.at[0])
            pltpu.sync_copy(rows_v.at[0], acc_sh.at[ei_v.at[base_cpt, 1]],
                            add=True)

        plsc.subcore_barrier()

        @pl.when(s < NS - 1)
        def _():
            pltpu.sync_copy(acc_sh.at[pl.ds(s * zrows, zrows)],
                            out_hbm.at[c, pl.ds(s * zrows, zrows)])

        @pl.when(s == NS - 1)
        def _():
            pltpu.sync_copy(acc_sh.at[pl.ds((NS - 1) * zrows, last)],
                            out_hbm.at[c, pl.ds((NS - 1) * zrows, last)])

    return agg


def _run(x, edge_index, W_l, b_l, W_r, W1, b1, W2, b2, interpret=False):
    n_nodes, d_in = x.shape
    n_edges = edge_index.shape[1]
    br = n_nodes

    ei3 = jnp.transpose(
        edge_index.astype(jnp.int32).reshape(2, n_edges // CHUNK, CHUNK),
        (1, 0, 2))

    w_cat = jnp.concatenate([W_l, W_r], axis=0).T
    grid = n_nodes // br
    y, r = pl.pallas_call(
        _proj_kernel,
        grid=(grid,),
        in_specs=[
            pl.BlockSpec((br, d_in), lambda i: (i, 0)),
            pl.BlockSpec((d_in, 2 * D), lambda i: (0, 0)),
        ],
        out_specs=[
            pl.BlockSpec((br, D), lambda i: (i, 0)),
            pl.BlockSpec((br, D), lambda i: (i, 0)),
        ],
        out_shape=[jax.ShapeDtypeStruct((n_nodes, D), jnp.float32)] * 2,
        interpret=interpret,
    )(x, w_cat)

    part = _make_agg(n_nodes, n_edges, interpret=interpret)(y, ei3)

    out = pl.pallas_call(
        _mlp_kernel,
        grid=(grid,),
        in_specs=[
            pl.BlockSpec((NC, br, D), lambda i: (0, i, 0)),
            pl.BlockSpec((br, D), lambda i: (i, 0)),
            pl.BlockSpec((1, D), lambda i: (0, 0)),
            pl.BlockSpec((D, D), lambda i: (0, 0)),
            pl.BlockSpec((1, D), lambda i: (0, 0)),
            pl.BlockSpec((D, D), lambda i: (0, 0)),
            pl.BlockSpec((1, D), lambda i: (0, 0)),
        ],
        out_specs=pl.BlockSpec((br, D), lambda i: (i, 0)),
        out_shape=jax.ShapeDtypeStruct((n_nodes, D), jnp.float32),
        interpret=interpret,
    )(part, r, b_l.reshape(1, D), W1.T,
      b1.reshape(1, D), W2.T, b2.reshape(1, D))
    return out


def kernel(x, edge_index, W_l, b_l, W_r, W1, b1, W2, b2):
    return _run(x, edge_index, W_l, b_l, W_r, W1, b1, W2, b2)


# speedup vs baseline: 25.7507x; 1.0253x over previous
import functools

import jax
import jax.numpy as jnp
from jax import lax
from jax.experimental import pallas as pl
from jax.experimental.pallas import tpu as pltpu
from jax.experimental.pallas import tpu_sc as plsc

D = 16
CHUNK = 128
NC = 2
NS = 16
NW = NC * NS
NBUF = 6
LA = 4


def _proj_kernel(x_ref, w_ref, h_ref):
    h_ref[...] = jnp.dot(x_ref[...], w_ref[...],
                         preferred_element_type=jnp.float32)


def _mlp_kernel(part_ref, h_in_ref, bl_ref, w1_ref, b1_ref, w2_ref, b2_ref,
                o_ref):
    h = part_ref[0] + part_ref[1] + bl_ref[...] + h_in_ref[:, D:]
    h = jnp.where(h >= 0, h, 0.01 * h)
    h = jnp.dot(h, w1_ref[...], preferred_element_type=jnp.float32) + b1_ref[...]
    h = jnp.where(h >= 0, h, 0.01 * h)
    o_ref[...] = jnp.dot(h, w2_ref[...], preferred_element_type=jnp.float32) + b2_ref[...]


def _make_agg(n_nodes, n_edges, interpret=False):
    acc_rows = ((n_nodes + 8 * NS - 1) // (8 * NS)) * (8 * NS)
    zrows = acc_rows // NS
    last = n_nodes - (NS - 1) * zrows
    assert 0 < last <= zrows
    assert n_edges % CHUNK == 0
    nrows = n_edges // CHUNK
    base_cpt = nrows // NW
    extra = nrows - base_cpt * NW
    ngrp = base_cpt // NBUF
    rest = base_cpt - ngrp * NBUF
    assert ngrp >= 1
    mesh = plsc.VectorSubcoreMesh(core_axis_name="c", subcore_axis_name="s",
                                  num_cores=NC, num_subcores=NS)

    @functools.partial(
        pl.kernel,
        out_type=jax.ShapeDtypeStruct((NC, n_nodes, D), jnp.float32),
        mesh=mesh,
        scratch_types=[
            pltpu.VMEM((base_cpt + 1, 2, CHUNK), jnp.int32),
            pltpu.VMEM((NBUF, CHUNK, D), jnp.float32),
            pltpu.VMEM((zrows, D), jnp.float32),
            pltpu.VMEM_SHARED((acc_rows, D), jnp.float32),
            [pltpu.SemaphoreType.DMA] * NBUF,
            [pltpu.SemaphoreType.DMA] * NBUF,
        ],
        compiler_params=pltpu.CompilerParams(use_tc_tiling_on_sc=False),
        interpret=interpret,
    )
    def agg(y_hbm, ei_hbm, out_hbm, ei_v, rows_v, zero_v, acc_sh,
            gsems, ssems):
        c = lax.axis_index("c")
        s = lax.axis_index("s")
        wid = s * NC + c
        start = wid * base_cpt + jnp.minimum(wid, extra)

        def zbody(i, carry):
            zero_v[i, :] = jnp.zeros((D,), jnp.float32)
            return carry

        lax.fori_loop(0, zrows, zbody, 0)
        pltpu.sync_copy(zero_v, acc_sh.at[pl.ds(s * zrows, zrows)])

        @pl.when(wid < extra)
        def _():
            pltpu.sync_copy(ei_hbm.at[pl.ds(start, base_cpt + 1)], ei_v)

        @pl.when(wid >= extra)
        def _():
            pltpu.sync_copy(ei_hbm.at[pl.ds(start, base_cpt)],
                            ei_v.at[pl.ds(0, base_cpt)])

        plsc.subcore_barrier()

        for b in range(LA):
            pltpu.async_copy(y_hbm.at[ei_v.at[b, 0]], rows_v.at[b], gsems[b])

        def step(j, b, bf):
            pltpu.make_async_copy(y_hbm.at[ei_v.at[j, 0]], rows_v.at[b],
                                  gsems[b]).wait()
            pltpu.async_copy(rows_v.at[b], acc_sh.at[ei_v.at[j, 1]],
                             ssems[b], add=True)
            f = j + LA

            @pl.when(f < base_cpt)
            def _():
                @pl.when(f >= NBUF)
                def _():
                    pltpu.make_async_copy(
                        rows_v.at[bf], acc_sh.at[ei_v.at[f - NBUF, 1]],
                        ssems[bf]).wait()

                pltpu.async_copy(y_hbm.at[ei_v.at[f, 0]], rows_v.at[bf],
                                 gsems[bf])

        def body(g, carry):
            base = g * NBUF
            for b in range(NBUF):
                step(base + b, b, (b + LA) % NBUF)
            return carry

        lax.fori_loop(0, ngrp, body, 0)
        for j in range(ngrp * NBUF, base_cpt):
            step(j, j % NBUF, (j + LA) % NBUF)
        for j in range(base_cpt - NBUF, base_cpt):
            b = j % NBUF
            pltpu.make_async_copy(rows_v.at[b], acc_sh.at[ei_v.at[j, 1]],
                                  ssems[b]).wait()

        @pl.when(wid < extra)
        def _():
            pltpu.sync_copy(y_hbm.at[ei_v.at[base_cpt, 0]], rows_v.at[0])
            pltpu.sync_copy(rows_v.at[0], acc_sh.at[ei_v.at[base_cpt, 1]],
                            add=True)

        plsc.subcore_barrier()

        @pl.when(s < NS - 1)
        def _():
            pltpu.sync_copy(acc_sh.at[pl.ds(s * zrows, zrows)],
                            out_hbm.at[c, pl.ds(s * zrows, zrows)])

        @pl.when(s == NS - 1)
        def _():
            pltpu.sync_copy(acc_sh.at[pl.ds((NS - 1) * zrows, last)],
                            out_hbm.at[c, pl.ds((NS - 1) * zrows, last)])

    return agg


def _run(x, edge_index, W_l, b_l, W_r, W1, b1, W2, b2, interpret=False):
    n_nodes, d_in = x.shape
    n_edges = edge_index.shape[1]
    br = n_nodes

    ei3 = jnp.transpose(
        edge_index.astype(jnp.int32).reshape(2, n_edges // CHUNK, CHUNK),
        (1, 0, 2))

    w_cat = jnp.concatenate([W_l, W_r], axis=0).T
    grid = n_nodes // br
    h = pl.pallas_call(
        _proj_kernel,
        grid=(grid,),
        in_specs=[
            pl.BlockSpec((br, d_in), lambda i: (i, 0)),
            pl.BlockSpec((d_in, 2 * D), lambda i: (0, 0)),
        ],
        out_specs=pl.BlockSpec((br, 2 * D), lambda i: (i, 0)),
        out_shape=jax.ShapeDtypeStruct((n_nodes, 2 * D), jnp.float32),
        interpret=interpret,
    )(x, w_cat)

    part = _make_agg(n_nodes, n_edges, interpret=interpret)(h[:, :D], ei3)

    out = pl.pallas_call(
        _mlp_kernel,
        grid=(grid,),
        in_specs=[
            pl.BlockSpec((NC, br, D), lambda i: (0, i, 0)),
            pl.BlockSpec((br, 2 * D), lambda i: (i, 0)),
            pl.BlockSpec((1, D), lambda i: (0, 0)),
            pl.BlockSpec((D, D), lambda i: (0, 0)),
            pl.BlockSpec((1, D), lambda i: (0, 0)),
            pl.BlockSpec((D, D), lambda i: (0, 0)),
            pl.BlockSpec((1, D), lambda i: (0, 0)),
        ],
        out_specs=pl.BlockSpec((br, D), lambda i: (i, 0)),
        out_shape=jax.ShapeDtypeStruct((n_nodes, D), jnp.float32),
        interpret=interpret,
    )(part, h, b_l.reshape(1, D), W1.T,
      b1.reshape(1, D), W2.T, b2.reshape(1, D))
    return out


def kernel(x, edge_index, W_l, b_l, W_r, W1, b1, W2, b2):
    return _run(x, edge_index, W_l, b_l, W_r, W1, b1, W2, b2)
